# Initial kernel scaffold; baseline (speedup 1.0000x reference)
import jax
import jax.numpy as jnp
from jax.experimental import pallas as pl


def kernel(idx, table):
    raise NotImplementedError("write your pallas kernel here")


# speedup vs baseline: 1.9747x; 1.9747x over previous
import functools

import jax
import jax.numpy as jnp
from jax import lax
from jax.experimental import pallas as pl
from jax.experimental.pallas import tpu as pltpu
from jax.experimental.pallas import tpu_sc as plsc

_NC = 2
_NS = 16
_NW = _NC * _NS

_V = 8192
_D = 8192
_BTOK = 8192
_BPW = _BTOK // _NW
_K = 4
_NBUF = 2
_NCHUNK = _BPW // _K


def _make_gather():
  mesh = plsc.VectorSubcoreMesh(
      core_axis_name="c", subcore_axis_name="s",
      num_cores=_NC, num_subcores=_NS)

  @functools.partial(
      pl.kernel,
      out_type=jax.ShapeDtypeStruct((_BTOK, _D), jnp.float32),
      mesh=mesh,
      scratch_types=[
          pltpu.VMEM((_NCHUNK, _K), jnp.int32),
          pltpu.VMEM((_K, _D), jnp.float32),
          pltpu.VMEM((_K, _D), jnp.float32),
          pltpu.SemaphoreType.DMA,
          pltpu.SemaphoreType.DMA,
          pltpu.SemaphoreType.DMA,
          pltpu.SemaphoreType.DMA,
      ],
  )
  def gather(table_hbm, idx_hbm, out_hbm,
             idx_v, buf0, buf1, in0, in1, out0, out1):
    bufs = (buf0, buf1)
    sem_in = (in0, in1)
    sem_out = (out0, out1)
    wid = lax.axis_index("s") * _NC + lax.axis_index("c")
    row0 = wid * _BPW

    pltpu.sync_copy(idx_hbm.at[wid], idx_v)

    for b in range(_NBUF):
      pltpu.async_copy(table_hbm.at[idx_v.at[b]], bufs[b], sem_in[b])

    @pl.loop(0, _NCHUNK, step=_NBUF)
    def _body(g):
      for b in range(_NBUF):
        c = g + b
        pltpu.make_async_copy(
            table_hbm.at[idx_v.at[c]], bufs[b], sem_in[b]).wait()
        out_slice = out_hbm.at[pl.ds(row0 + c * _K, _K)]
        pltpu.async_copy(bufs[b], out_slice, sem_out[b])

        @pl.when(c + _NBUF < _NCHUNK)
        def _refill():
          pltpu.make_async_copy(bufs[b], out_slice, sem_out[b]).wait()
          pltpu.async_copy(
              table_hbm.at[idx_v.at[c + _NBUF]], bufs[b], sem_in[b])

    for b in range(_NBUF):
      c = _NCHUNK - _NBUF + b
      pltpu.make_async_copy(
          bufs[b], out_hbm.at[pl.ds(row0 + c * _K, _K)], sem_out[b]).wait()

  return gather


_gather = _make_gather()


def kernel(idx, table):
  b, t = idx.shape
  idx3 = idx.astype(jnp.int32).reshape(_NW, _NCHUNK, _K)
  out = _gather(table, idx3)
  return out.reshape(b, t, _V)
